# baseline (speedup 1.0000x reference)
import jax
import jax.numpy as jnp
from jax import lax
from jax.experimental import pallas as pl
from jax.experimental.pallas import tpu as pltpu
from jax.experimental.pallas import tpu_sc as plsc

N = 10000
D = 128
H = 64
E = 320000
NEG = 0.2

NC = 2
NS = 16
NW = NC * NS
SB = 128
NB = 6
CH = SB * NB
CPW = 14
EPW = CH * CPW
EPAD = NW * EPW
NA = 10240
RPS = NA // NS
HP = 80
NAD = 10016


def _sc_edge_kernel(gpack, adpack, idx):
    mesh = plsc.VectorSubcoreMesh(core_axis_name="c", subcore_axis_name="s")
    kern = pl.kernel(
        _edge_body,
        out_type=jax.ShapeDtypeStruct((NC, NA, HP), jnp.float32),
        mesh=mesh,
        scratch_types=[
            pltpu.VMEM((2 * NB, SB), jnp.int32),
            pltpu.VMEM((NB, SB, 16), jnp.float32),
            pltpu.VMEM((NB, SB, HP), jnp.float32),
            pltpu.VMEM_SHARED((NA, HP), jnp.float32),
            pltpu.SemaphoreType.DMA,
            pltpu.SemaphoreType.DMA,
        ],
        compiler_params=pltpu.CompilerParams(use_tc_tiling_on_sc=False),
    )
    return kern(gpack, adpack, idx)


def _edge_body(gpack, adpack, idx, acc_out,
               idxv, adstv, hrows, acc_s, semg, sems):
    cid = lax.axis_index("c")
    sid = lax.axis_index("s")
    wid = cid * NS + sid

    @pl.loop(0, SB)
    def _z(r):
        for q in range(HP // 16):
            hrows[0, r, pl.ds(q * 16, 16)] = jnp.zeros((16,), jnp.float32)

    for j in range(RPS // SB):
        pltpu.sync_copy(hrows.at[0], acc_s.at[pl.ds(sid * RPS + j * SB, SB)])
    plsc.subcore_barrier()

    @pl.loop(0, CPW)
    def _chunk(i):
        pltpu.sync_copy(idx.at[wid, i], idxv)
        gat = []
        for b in range(NB):
            gat.append(pltpu.async_copy(gpack.at[idxv.at[b]],
                                        hrows.at[b], semg))
            gat.append(pltpu.async_copy(adpack.at[idxv.at[NB + b]],
                                        adstv.at[b], semg))
        for d in gat:
            d.wait()

        for b in range(NB):
            @pl.loop(0, SB // 16)
            def _row16(t, b=b):
                for k in range(16):
                    r = t * 16 + k
                    av = hrows[b, r, pl.ds(H, 16)] + adstv[b, r]
                    av = jnp.maximum(av, NEG * av)
                    w = jnp.exp(av)
                    hrows[b, r, pl.ds(H, 16)] = w
                    for q in range(H // 16):
                        hrows[b, r, pl.ds(q * 16, 16)] = (
                            hrows[b, r, pl.ds(q * 16, 16)] * w)

        sca = []
        for b in range(NB):
            sca.append(pltpu.async_copy(hrows.at[b],
                                        acc_s.at[idxv.at[NB + b]],
                                        sems, add=True))
        for d in sca:
            d.wait()

    plsc.subcore_barrier()
    pltpu.sync_copy(acc_s.at[pl.ds(sid * RPS, RPS)],
                    acc_out.at[cid, pl.ds(sid * RPS, RPS)])


def _tc0_body(x_ref, wn_ref, bn_ref, wg_ref, asw_ref, adw_ref,
              g_ref, av_ref):
    h = jnp.dot(x_ref[...], wn_ref[...],
                preferred_element_type=jnp.float32) + bn_ref[...]
    gm = jnp.dot(h, wg_ref[...], preferred_element_type=jnp.float32)
    av16 = jnp.dot(gm, asw_ref[...], preferred_element_type=jnp.float32)
    g_ref[...] = jnp.concatenate([gm, av16], axis=1)
    av_ref[...] = jnp.dot(gm, adw_ref[...], preferred_element_type=jnp.float32)


def _tc0(x, Wn, bn, Wg, asw, adw):
    return pl.pallas_call(
        _tc0_body,
        out_shape=[jax.ShapeDtypeStruct((N, HP), jnp.float32),
                   jax.ShapeDtypeStruct((N, 16), jnp.float32)],
    )(x, Wn, bn.reshape(1, H), Wg, asw, adw)


def _tcmid_body(a0_ref, a1_ref, b_ref, wg_ref, asw_ref, adw_ref,
                g_ref, av_ref):
    den = a0_ref[:, pl.ds(H, 1)] + a1_ref[:, pl.ds(H, 1)] + 1e-16
    h = (a0_ref[:, :H] + a1_ref[:, :H]) / den + b_ref[...]
    h = jnp.maximum(h, 0.0)
    gm = jnp.dot(h, wg_ref[...], preferred_element_type=jnp.float32)
    av16 = jnp.dot(gm, asw_ref[...], preferred_element_type=jnp.float32)
    g_ref[...] = jnp.concatenate([gm, av16], axis=1)
    av_ref[...] = jnp.dot(gm, adw_ref[...], preferred_element_type=jnp.float32)


def _tcmid(a0, a1, b, Wg, asw, adw):
    return pl.pallas_call(
        _tcmid_body,
        out_shape=[jax.ShapeDtypeStruct((N, HP), jnp.float32),
                   jax.ShapeDtypeStruct((N, 16), jnp.float32)],
    )(a0, a1, b, Wg, asw, adw)


def _tcfin_body(a0_ref, a1_ref, b_ref,
                wp1a_ref, wp1b_ref, bp1_ref, wp2_ref, bp2_ref,
                wc1_ref, bc1_ref, wc2_ref, bc2_ref,
                wt1_ref, bt1_ref, wt2_ref, bt2_ref,
                sc_ref, ty_ref, ge_ref):
    den = a0_ref[:, pl.ds(H, 1)] + a1_ref[:, pl.ds(H, 1)] + 1e-16
    h = (a0_ref[:, :H] + a1_ref[:, :H]) / den + b_ref[...]
    gmean = jnp.sum(h, axis=0, keepdims=True) * (1.0 / N)
    gmax = jnp.max(h, axis=0, keepdims=True)
    pre = (jnp.dot(gmean, wp1a_ref[...], preferred_element_type=jnp.float32)
           + jnp.dot(gmax, wp1b_ref[...], preferred_element_type=jnp.float32)
           + bp1_ref[...])
    ge = jnp.dot(jnp.maximum(pre, 0.0), wp2_ref[...],
                 preferred_element_type=jnp.float32) + bp2_ref[...]
    ge_ref[...] = ge
    c = jnp.dot(jnp.maximum(
        jnp.dot(ge, wc1_ref[...], preferred_element_type=jnp.float32)
        + bc1_ref[...], 0.0), wc2_ref[...],
        preferred_element_type=jnp.float32) + bc2_ref[...]
    sc_ref[...] = 1.0 / (1.0 + jnp.exp(-c))
    ty_ref[...] = jnp.dot(jnp.maximum(
        jnp.dot(ge, wt1_ref[...], preferred_element_type=jnp.float32)
        + bt1_ref[...], 0.0), wt2_ref[...],
        preferred_element_type=jnp.float32) + bt2_ref[...]


def _tcfin(a0, a1, b, Wp1, bp1, Wp2, bp2,
           Wc1, bc1, Wc2, bc2, Wt1, bt1, Wt2, bt2):
    return pl.pallas_call(
        _tcfin_body,
        out_shape=[jax.ShapeDtypeStruct((1, 1), jnp.float32),
                   jax.ShapeDtypeStruct((1, 6), jnp.float32),
                   jax.ShapeDtypeStruct((1, H // 2), jnp.float32)],
    )(a0, a1, b,
      Wp1[:H], Wp1[H:], bp1.reshape(1, H), Wp2, bp2.reshape(1, H // 2),
      Wc1, bc1.reshape(1, H // 4), Wc2, bc2.reshape(1, 1),
      Wt1, bt1.reshape(1, H // 4), Wt2, bt2.reshape(1, 6))


def _layer_edges(edge_index):
    loop = jnp.arange(N, dtype=jnp.int32)
    pad = EPAD - (E + N)
    srcs = jnp.concatenate([edge_index[0], loop,
                            jnp.zeros((pad,), jnp.int32)])
    dsts = jnp.concatenate([edge_index[1], loop,
                            jnp.full((pad,), N, jnp.int32)])
    s4 = srcs.reshape(NW, CPW, NB, SB)
    d4 = dsts.reshape(NW, CPW, NB, SB)
    return jnp.concatenate([s4, d4], axis=2)


def _sc_layer(gpack, adv16, idx):
    adpack = jnp.concatenate(
        [adv16, jnp.zeros((NAD - N, 16), jnp.float32)])
    acc2 = _sc_edge_kernel(gpack, adpack, idx)
    return acc2[0, :N, :], acc2[1, :N, :]


def _aw16(a):
    return jnp.broadcast_to(a[0, 0][:, None], (H, 16))


def kernel(x, edge_index, edge_attr, Wn, bn, Wg0, as0, ad0, bg0,
           Wg1, as1, ad1, bg1, Wg2, as2, ad2, bg2,
           Wp1, bp1, Wp2, bp2, Wc1, bc1, Wc2, bc2, Wt1, bt1, Wt2, bt2):
    idx = _layer_edges(edge_index)

    g0, av0 = _tc0(x, Wn, bn, Wg0, _aw16(as0), _aw16(ad0))
    a0, a1 = _sc_layer(g0, av0, idx)

    g1, av1 = _tcmid(a0, a1, bg0.reshape(1, H), Wg1, _aw16(as1), _aw16(ad1))
    a0, a1 = _sc_layer(g1, av1, idx)

    g2, av2 = _tcmid(a0, a1, bg1.reshape(1, H), Wg2, _aw16(as2), _aw16(ad2))
    a0, a1 = _sc_layer(g2, av2, idx)

    scores, types, ge = _tcfin(a0, a1, bg2.reshape(1, H),
                               Wp1, bp1, Wp2, bp2, Wc1, bc1, Wc2, bc2,
                               Wt1, bt1, Wt2, bt2)
    return (scores, types, ge)
